# baseline (speedup 1.0000x reference)
import functools

import jax
import jax.numpy as jnp
from jax import lax
from jax.experimental import pallas as pl
from jax.experimental.pallas import tpu as pltpu
from jax.experimental.pallas import tpu_sc as plsc

_EPS = 1e-5


def _stem_body(x_ref, W1_ref, b1_ref, g1_ref, be1_ref, WbT_ref, WdT_ref,
               bg_ref, fncm_ref, fnT_ref, y_ref, c_ref):
    x = x_ref[...]
    W1 = W1_ref[...]
    Bn = x.shape[0]
    h = jnp.stack([jnp.dot(W1, x[b], preferred_element_type=jnp.float32)
                   for b in range(Bn)])
    h = h + b1_ref[...][None, :, None]
    mean = jnp.mean(h, axis=(0, 2), keepdims=True)
    var = jnp.mean((h - mean) ** 2, axis=(0, 2), keepdims=True)
    feat = (h - mean) / jnp.sqrt(var + _EPS)
    feat = feat * g1_ref[...][None, :, None] + be1_ref[...][None, :, None]
    nrm = jnp.sqrt(jnp.sum(feat * feat, axis=1, keepdims=True))
    fn = feat / jnp.maximum(nrm, 1e-12)
    fncm_ref[...] = fn
    fnT_ref[...] = jnp.stack([fn[b].T for b in range(Bn)])
    featT = jnp.stack([feat[b].T for b in range(Bn)])
    WbT = WbT_ref[...]
    WdT = WdT_ref[...]
    y_ref[...] = jnp.stack(
        [jnp.dot(featT[b], WbT, preferred_element_type=jnp.float32)
         for b in range(Bn)])
    c_ref[...] = jnp.stack(
        [jnp.dot(featT[b], WdT, preferred_element_type=jnp.float32)
         for b in range(Bn)]) + bg_ref[...][None, None, :]


def _knn_body(fnT_ref, fncm_ref, nn_ref, *, N, K):
    b = pl.program_id(0)
    A = fnT_ref[0]
    Bm = fncm_ref[0]
    S = jnp.dot(A, Bm, preferred_element_type=jnp.float32)
    sqi = jnp.sum(A * A, axis=1)
    sqj = jnp.sum(Bm * Bm, axis=0)
    d = sqi[:, None] - 2.0 * S + sqj[None, :]
    iof = lax.broadcasted_iota(jnp.int32, d.shape, 1).astype(jnp.float32)
    nbig = jnp.float32(N)
    inf = jnp.float32(jnp.inf)
    cols = []
    for _ in range(K):
        m = jnp.min(d, axis=1, keepdims=True)
        eq = d == m
        idxf = jnp.min(jnp.where(eq, iof, nbig), axis=1)
        cols.append(idxf)
        d = jnp.where(eq, inf, d)
    nn_ref[0] = jnp.stack(cols, axis=1).astype(jnp.int32) + b * N


def _make_gather_max(BN, Dh, Dp, K, CH, IW):
    info = plsc.get_sparse_core_info()
    NC, NS = info.num_cores, info.num_subcores
    NW = NC * NS
    rows_w = BN // NW
    nchunks = rows_w // CH
    mesh = plsc.VectorSubcoreMesh(core_axis_name="c", subcore_axis_name="s")

    @functools.partial(
        pl.kernel, mesh=mesh,
        out_type=jax.ShapeDtypeStruct((BN, Dh), jnp.float32),
        scratch_types=[
            pltpu.VMEM((nchunks, IW), jnp.int32),
            pltpu.VMEM((2, IW, Dp), jnp.float32),
            pltpu.VMEM((2, CH, Dh), jnp.float32),
            pltpu.SemaphoreType.DMA,
            pltpu.SemaphoreType.DMA,
        ],
    )
    def gather_max(y_hbm, idx_hbm, out_hbm, idx_v, rows_v, outr, sem_g, sem_o):
        wid = lax.axis_index("s") * NC + lax.axis_index("c")
        base = wid * rows_w
        pltpu.sync_copy(idx_hbm.at[wid], idx_v)
        pltpu.async_copy(y_hbm.at[idx_v.at[0]], rows_v.at[0], sem_g)

        def chunk(ci, carry):
            buf = lax.rem(ci, 2)
            nxt = ci + 1

            @pl.when(nxt < nchunks)
            def _():
                pltpu.async_copy(y_hbm.at[idx_v.at[nxt]],
                                 rows_v.at[lax.rem(nxt, 2)], sem_g)

            pltpu.make_async_copy(y_hbm.at[idx_v.at[ci]],
                                  rows_v.at[buf], sem_g).wait()

            @pl.when(ci >= 2)
            def _():
                pltpu.make_async_copy(
                    outr.at[buf], out_hbm.at[pl.ds(base, CH)], sem_o).wait()

            for n in range(CH):
                for s in range(Dh // 16):
                    sl = pl.ds(s * 16, 16)
                    acc = rows_v[buf, n * K, sl]
                    for j in range(1, K):
                        acc = jnp.maximum(acc, rows_v[buf, n * K + j, sl])
                    outr[buf, n, sl] = acc
            pltpu.async_copy(outr.at[buf],
                             out_hbm.at[pl.ds(base + ci * CH, CH)], sem_o)
            return carry

        lax.fori_loop(0, nchunks, chunk, 0)
        pltpu.make_async_copy(outr.at[0], out_hbm.at[pl.ds(base, CH)],
                              sem_o).wait()
        pltpu.make_async_copy(outr.at[1], out_hbm.at[pl.ds(base, CH)],
                              sem_o).wait()

    return gather_max


def _head_body(maxy_ref, c_ref, gg_ref, beg_ref, W2T_ref, b2_ref, g2_ref,
               be2_ref, x_ref, out_ref):
    e = c_ref[...] + maxy_ref[...]
    mean = jnp.mean(e, axis=0, keepdims=True)
    var = jnp.mean((e - mean) ** 2, axis=0, keepdims=True)
    eh = (e - mean) / jnp.sqrt(var + _EPS)
    eh = eh * gg_ref[...][None, :] + beg_ref[...][None, :]
    g = 0.5 * eh * (1.0 + lax.erf(eh * (2.0 ** -0.5)))
    o = jnp.dot(g, W2T_ref[...], preferred_element_type=jnp.float32)
    o = o + b2_ref[...][None, :]
    mean2 = jnp.mean(o, axis=0, keepdims=True)
    var2 = jnp.mean((o - mean2) ** 2, axis=0, keepdims=True)
    oh = (o - mean2) / jnp.sqrt(var2 + _EPS)
    oh = oh * g2_ref[...][None, :] + be2_ref[...][None, :]
    x = x_ref[...]
    Bn, Cn, Nn = x.shape
    o4 = oh.reshape(Bn, Nn, Cn)
    out_ref[...] = jnp.stack([o4[b].T for b in range(Bn)]) + x


def kernel(x, W1, b1, g1, be1, Wg, bg, gg, beg, W2, b2, g2, be2):
    B, C, H, W = x.shape
    N = H * W
    BN = B * N
    hid = Wg.shape[0]
    K = 9
    R = 224
    x2 = x.reshape(B, C, N)
    Dp = 256
    WbT = jnp.pad(Wg[:, C:].T, ((0, 0), (0, Dp - hid)))
    WdT = (Wg[:, :C] - Wg[:, C:]).T

    fncm, fnT, y, cpre = pl.pallas_call(
        _stem_body,
        out_shape=[
            jax.ShapeDtypeStruct((B, C, N), jnp.float32),
            jax.ShapeDtypeStruct((B, N, C), jnp.float32),
            jax.ShapeDtypeStruct((B, N, Dp), jnp.float32),
            jax.ShapeDtypeStruct((B, N, hid), jnp.float32),
        ],
    )(x2, W1, b1, g1, be1, WbT, WdT, bg)

    nn = pl.pallas_call(
        functools.partial(_knn_body, N=N, K=K),
        grid=(B, N // R),
        in_specs=[
            pl.BlockSpec((1, R, C), lambda b, j: (b, j, 0)),
            pl.BlockSpec((1, C, N), lambda b, j: (b, 0, 0)),
        ],
        out_specs=pl.BlockSpec((1, R, K), lambda b, j: (b, j, 0)),
        out_shape=jax.ShapeDtypeStruct((B, N, K), jnp.int32),
    )(fnT, fncm)

    info = plsc.get_sparse_core_info()
    NW = info.num_cores * info.num_subcores
    CH = 8
    maxy = _make_gather_max(BN, hid, Dp, K, CH, CH * K)(
        y.reshape(BN, Dp), nn.reshape(NW, (BN // NW) // CH, CH * K))

    out = pl.pallas_call(
        _head_body,
        out_shape=jax.ShapeDtypeStruct((B, C, N), jnp.float32),
    )(maxy, cpre.reshape(BN, hid), gg, beg, W2.T, b2, g2, be2, x2)
    return out.reshape(B, C, N, 1)


# speedup vs baseline: 1.0594x; 1.0594x over previous
import functools

import jax
import jax.numpy as jnp
from jax import lax
from jax.experimental import pallas as pl
from jax.experimental.pallas import tpu as pltpu
from jax.experimental.pallas import tpu_sc as plsc

_EPS = 1e-5


def _stem_body(x_ref, W1_ref, b1_ref, g1_ref, be1_ref, WbT_ref, WdT_ref,
               bg_ref, fncm_ref, fnT_ref, y_ref, c_ref):
    x = x_ref[...]
    W1 = W1_ref[...]
    Bn = x.shape[0]
    h = jnp.stack([jnp.dot(W1, x[b], preferred_element_type=jnp.float32)
                   for b in range(Bn)])
    h = h + b1_ref[...][None, :, None]
    mean = jnp.mean(h, axis=(0, 2), keepdims=True)
    var = jnp.mean((h - mean) ** 2, axis=(0, 2), keepdims=True)
    feat = (h - mean) / jnp.sqrt(var + _EPS)
    feat = feat * g1_ref[...][None, :, None] + be1_ref[...][None, :, None]
    nrm = jnp.sqrt(jnp.sum(feat * feat, axis=1, keepdims=True))
    fn = feat / jnp.maximum(nrm, 1e-12)
    fncm_ref[...] = fn
    fnT_ref[...] = jnp.stack([fn[b].T for b in range(Bn)])
    featT = jnp.stack([feat[b].T for b in range(Bn)])
    WbT = WbT_ref[...]
    WdT = WdT_ref[...]
    y_ref[...] = jnp.stack(
        [jnp.dot(featT[b], WbT, preferred_element_type=jnp.float32)
         for b in range(Bn)])
    c_ref[...] = jnp.stack(
        [jnp.dot(featT[b], WdT, preferred_element_type=jnp.float32)
         for b in range(Bn)]) + bg_ref[...][None, None, :]


def _knn_body(fnT_ref, fncm_ref, nn_ref, *, N, K):
    b = pl.program_id(0)
    A = fnT_ref[0]
    Bm = fncm_ref[0]
    S = jnp.dot(A, Bm, preferred_element_type=jnp.float32)
    sqi = jnp.sum(A * A, axis=1)
    sqj = jnp.sum(Bm * Bm, axis=0)
    d = sqi[:, None] - 2.0 * S + sqj[None, :]
    iof = lax.broadcasted_iota(jnp.int32, d.shape, 1).astype(jnp.float32)
    nbig = jnp.float32(N)
    inf = jnp.float32(jnp.inf)
    cols = []
    for _ in range(K):
        m = jnp.min(d, axis=1, keepdims=True)
        eq = d == m
        idxf = jnp.min(jnp.where(eq, iof, nbig), axis=1)
        cols.append(idxf)
        d = jnp.where(eq, inf, d)
    nn_ref[0] = jnp.stack(cols, axis=1).astype(jnp.int32) + b * N


def _make_gather_max(BN, Dh, Dp, K, CH, IW):
    info = plsc.get_sparse_core_info()
    NC, NS = info.num_cores, info.num_subcores
    NW = NC * NS
    rows_w = BN // NW
    nchunks = rows_w // CH
    mesh = plsc.VectorSubcoreMesh(core_axis_name="c", subcore_axis_name="s")

    @functools.partial(
        pl.kernel, mesh=mesh,
        out_type=jax.ShapeDtypeStruct((BN, Dh), jnp.float32),
        scratch_types=[
            pltpu.VMEM((nchunks, IW), jnp.int32),
            pltpu.VMEM((2, IW, Dp), jnp.float32),
            pltpu.VMEM((2, CH, Dh), jnp.float32),
            pltpu.SemaphoreType.DMA,
            pltpu.SemaphoreType.DMA,
        ],
    )
    def gather_max(y_hbm, idx_hbm, out_hbm, idx_v, rows_v, outr, sem_g, sem_o):
        wid = lax.axis_index("s") * NC + lax.axis_index("c")
        base = wid * rows_w
        pltpu.sync_copy(idx_hbm.at[wid], idx_v)
        pltpu.async_copy(y_hbm.at[idx_v.at[0]], rows_v.at[0], sem_g)

        def chunk(ci, carry):
            buf = lax.rem(ci, 2)
            nxt = ci + 1

            @pl.when(nxt < nchunks)
            def _():
                pltpu.async_copy(y_hbm.at[idx_v.at[nxt]],
                                 rows_v.at[lax.rem(nxt, 2)], sem_g)

            pltpu.make_async_copy(y_hbm.at[idx_v.at[ci]],
                                  rows_v.at[buf], sem_g).wait()

            @pl.when(ci >= 2)
            def _():
                pltpu.make_async_copy(
                    outr.at[buf], out_hbm.at[pl.ds(base, CH)], sem_o).wait()

            for n in range(CH):
                for s in range(Dh // 16):
                    sl = pl.ds(s * 16, 16)
                    acc = rows_v[buf, n * K, sl]
                    for j in range(1, K):
                        acc = jnp.maximum(acc, rows_v[buf, n * K + j, sl])
                    outr[buf, n, sl] = acc
            pltpu.async_copy(outr.at[buf],
                             out_hbm.at[pl.ds(base + ci * CH, CH)], sem_o)
            return carry

        lax.fori_loop(0, nchunks, chunk, 0)
        pltpu.make_async_copy(outr.at[0], out_hbm.at[pl.ds(base, CH)],
                              sem_o).wait()
        pltpu.make_async_copy(outr.at[1], out_hbm.at[pl.ds(base, CH)],
                              sem_o).wait()

    return gather_max


def _head_body(maxy_ref, c_ref, gg_ref, beg_ref, W2T_ref, b2_ref, g2_ref,
               be2_ref, x_ref, out_ref):
    e = c_ref[...] + maxy_ref[...]
    mean = jnp.mean(e, axis=0, keepdims=True)
    var = jnp.mean((e - mean) ** 2, axis=0, keepdims=True)
    eh = (e - mean) / jnp.sqrt(var + _EPS)
    eh = eh * gg_ref[...][None, :] + beg_ref[...][None, :]
    g = 0.5 * eh * (1.0 + lax.erf(eh * (2.0 ** -0.5)))
    o = jnp.dot(g, W2T_ref[...], preferred_element_type=jnp.float32)
    o = o + b2_ref[...][None, :]
    mean2 = jnp.mean(o, axis=0, keepdims=True)
    var2 = jnp.mean((o - mean2) ** 2, axis=0, keepdims=True)
    oh = (o - mean2) / jnp.sqrt(var2 + _EPS)
    oh = oh * g2_ref[...][None, :] + be2_ref[...][None, :]
    x = x_ref[...]
    Bn, Cn, Nn = x.shape
    o4 = oh.reshape(Bn, Nn, Cn)
    out_ref[...] = jnp.stack([o4[b].T for b in range(Bn)]) + x


def kernel(x, W1, b1, g1, be1, Wg, bg, gg, beg, W2, b2, g2, be2):
    B, C, H, W = x.shape
    N = H * W
    BN = B * N
    hid = Wg.shape[0]
    K = 9
    R = 448
    x2 = x.reshape(B, C, N)
    Dp = 256
    WbT = jnp.pad(Wg[:, C:].T, ((0, 0), (0, Dp - hid)))
    WdT = (Wg[:, :C] - Wg[:, C:]).T

    fncm, fnT, y, cpre = pl.pallas_call(
        _stem_body,
        out_shape=[
            jax.ShapeDtypeStruct((B, C, N), jnp.float32),
            jax.ShapeDtypeStruct((B, N, C), jnp.float32),
            jax.ShapeDtypeStruct((B, N, Dp), jnp.float32),
            jax.ShapeDtypeStruct((B, N, hid), jnp.float32),
        ],
    )(x2, W1, b1, g1, be1, WbT, WdT, bg)

    nn = pl.pallas_call(
        functools.partial(_knn_body, N=N, K=K),
        grid=(B, N // R),
        in_specs=[
            pl.BlockSpec((1, R, C), lambda b, j: (b, j, 0)),
            pl.BlockSpec((1, C, N), lambda b, j: (b, 0, 0)),
        ],
        out_specs=pl.BlockSpec((1, R, K), lambda b, j: (b, j, 0)),
        out_shape=jax.ShapeDtypeStruct((B, N, K), jnp.int32),
    )(fnT, fncm)

    info = plsc.get_sparse_core_info()
    NW = info.num_cores * info.num_subcores
    CH = 8
    maxy = _make_gather_max(BN, hid, Dp, K, CH, CH * K)(
        y.reshape(BN, Dp), nn.reshape(NW, (BN // NW) // CH, CH * K))

    out = pl.pallas_call(
        _head_body,
        out_shape=jax.ShapeDtypeStruct((B, C, N), jnp.float32),
    )(maxy, cpre.reshape(BN, hid), gg, beg, W2.T, b2, g2, be2, x2)
    return out.reshape(B, C, N, 1)
